# baseline (speedup 1.0000x reference)
import functools

import jax
import jax.numpy as jnp
from jax import lax
from jax.experimental import pallas as pl
from jax.experimental.pallas import tpu as pltpu
from jax.experimental.pallas import tpu_sc as plsc

PH = 16
DH = 1024
NROWS = PH * PH
D = 2 * DH
NC = 2
HALF = NROWS // NC
NGRP = HALF // PH

_mesh = plsc.ScalarSubcoreMesh(axis_name="c", num_cores=NC)


@functools.partial(
    pl.kernel,
    mesh=_mesh,
    out_type=jax.ShapeDtypeStruct((NROWS, D), jnp.float32),
    scratch_types=[
        pltpu.MemorySpace.VMEM_SHARED((PH, DH), jnp.float32),
        pltpu.MemorySpace.VMEM_SHARED((PH, DH), jnp.float32),
        pltpu.MemorySpace.VMEM_SHARED((HALF, D), jnp.float32),
        pltpu.SemaphoreType.DMA,
        pltpu.SemaphoreType.DMA,
    ],
)
def _spatial_pos_enc(
    row_hbm, col_hbm, out_hbm, row_sp, col_sp, drain_sp, sem_i, sem_o
):
    half = lax.axis_index("c")
    r0 = half * HALF
    i0 = half * NGRP

    in_c = pltpu.async_copy(col_hbm, col_sp, sem_i)
    in_r = pltpu.async_copy(row_hbm, row_sp, sem_i)

    in_c.wait()

    def colg(g, carry):
        pltpu.async_copy(
            col_sp, out_hbm.at[pl.ds(r0 + g * PH, PH), pl.ds(DH, DH)], sem_o
        )
        return carry

    lax.fori_loop(0, NGRP, colg, 0)

    in_r.wait()

    def rowt(t, carry):
        pltpu.async_copy(
            row_sp.at[pl.ds(i0 + t // PH, 1)],
            out_hbm.at[pl.ds(r0 + t, 1), pl.ds(0, DH)],
            sem_o,
        )
        return carry

    lax.fori_loop(0, HALF, rowt, 0)

    pltpu.make_async_copy(out_hbm.at[pl.ds(r0, HALF)], drain_sp, sem_o).wait()


def kernel(row_embed, col_embed):
    return _spatial_pos_enc(row_embed, col_embed)
